# baseline (speedup 1.0000x reference)
import jax
import jax.numpy as jnp

N = 10000
HID = 128


def _gru_cell(x, h, w_ih, w_hh, b_ih, b_hh):
    gi = x @ w_ih.T + b_ih
    gh = h @ w_hh.T + b_hh
    i_r, i_z, i_n = jnp.split(gi, 3, axis=-1)
    h_r, h_z, h_n = jnp.split(gh, 3, axis=-1)
    r = jax.nn.sigmoid(i_r + h_r)
    z = jax.nn.sigmoid(i_z + h_z)
    n = jnp.tanh(i_n + r * h_n)
    return (1.0 - z) * n + z * h


def _gnn(feat, src, dst, W1, b1, W2, b2):
    m1 = feat @ W1.T + b1
    h1 = jax.ops.segment_sum(m1[src], dst, num_segments=N)
    h1 = jax.nn.relu(h1)
    m2 = h1 @ W2.T + b2
    h2 = jax.ops.segment_sum(m2[src], dst, num_segments=N)
    return h2


def kernel(t, x0, edge_index, gru_w_ih, gru_w_hh, gru_b_ih, gru_b_hh, W1, b1, W2, b2):
    src = edge_index[0]
    dst = edge_index[1]
    Tn = t.shape[0]
    outs = [x0]
    douts = []
    h = jnp.zeros((x0.shape[0], HID), dtype=x0.dtype)
    xi = _gru_cell(x0, h, gru_w_ih, gru_w_hh, gru_b_ih, gru_b_hh)
    h = xi
    xii = _gnn(xi, src, dst, W1, b1, W2, b2)
    douts.append(xii)
    dt = t[1] - t[0]
    temp = x0 + dt * xii
    outs.append(temp)
    for i in range(2, Tn):
        xi = _gru_cell(temp, h, gru_w_ih, gru_w_hh, gru_b_ih, gru_b_hh)
        h = xi
        xii = _gnn(xi, src, dst, W1, b1, W2, b2)
        douts.append(xii)
        dt = t[i] - t[i - 1]
        temp = temp + dt * xii
        outs.append(temp)
    xi = _gru_cell(temp, h, gru_w_ih, gru_w_hh, gru_b_ih, gru_b_hh)
    xii = _gnn(xi, src, dst, W1, b1, W2, b2)
    douts.append(xii)
    output = jnp.stack(outs, axis=0)
    doutput = jnp.stack(douts, axis=0)
    return jnp.concatenate([output, doutput], axis=-1)


# speedup vs baseline: 6.9160x; 6.9160x over previous
import functools

import jax
import jax.numpy as jnp
from jax import lax
from jax.experimental import pallas as pl
from jax.experimental.pallas import tpu as pltpu
from jax.experimental.pallas import tpu_sc as plsc

N = 10000
HID = 128
T = 8
E = 320000

NC = 2
NS = 16
EDGES_PER_W = E // (NC * NS)
CHUNK = 40
NCHUNK = EDGES_PER_W // CHUNK
NSTAGE = 5
SCHUNK = NCHUNK // NSTAGE
ROWS_PER_S = N // NS

BLK = 2000
F32 = jnp.float32


def _segsum_body(m_hbm, srcr, dstr, zeros_hbm, out_hbm,
                 src_v, dst_v, rows_a, rows_b, acc, sem_a, sem_b):
    cid = lax.axis_index("c")
    sid = lax.axis_index("s")

    pltpu.sync_copy(zeros_hbm, acc.at[pl.ds(sid * ROWS_PER_S, ROWS_PER_S)])
    plsc.subcore_barrier()

    def stage(k, carry):
        pltpu.sync_copy(srcr.at[cid, sid, k], src_v)
        pltpu.sync_copy(dstr.at[cid, sid, k], dst_v)
        pltpu.make_async_copy(m_hbm.at[src_v.at[0]], rows_a, sem_a).start()

        def body(i, carry2):
            c0 = 2 * i
            c1 = c0 + 1
            pltpu.make_async_copy(m_hbm.at[src_v.at[c1]], rows_b,
                                  sem_b).start()
            pltpu.make_async_copy(m_hbm.at[src_v.at[c0]], rows_a,
                                  sem_a).wait()
            pltpu.sync_copy(rows_a, acc.at[dst_v.at[c0]], add=True)

            @pl.when(i < SCHUNK // 2 - 1)
            def _():
                pltpu.make_async_copy(m_hbm.at[src_v.at[c0 + 2]], rows_a,
                                      sem_a).start()

            pltpu.make_async_copy(m_hbm.at[src_v.at[c1]], rows_b,
                                  sem_b).wait()
            pltpu.sync_copy(rows_b, acc.at[dst_v.at[c1]], add=True)
            return carry2

        lax.fori_loop(0, SCHUNK // 2, body, 0)
        return carry

    lax.fori_loop(0, NSTAGE, stage, 0)

    plsc.subcore_barrier()
    pltpu.sync_copy(acc.at[pl.ds(sid * ROWS_PER_S, ROWS_PER_S)],
                    out_hbm.at[cid, sid])


_segsum_call = functools.partial(
    pl.kernel,
    mesh=plsc.VectorSubcoreMesh(core_axis_name="c", subcore_axis_name="s"),
    out_type=jax.ShapeDtypeStruct((NC, NS, ROWS_PER_S, HID), F32),
    scratch_types=[
        pltpu.VMEM((SCHUNK, CHUNK), jnp.int32),
        pltpu.VMEM((SCHUNK, CHUNK), jnp.int32),
        pltpu.VMEM((CHUNK, HID), F32),
        pltpu.VMEM((CHUNK, HID), F32),
        pltpu.VMEM_SHARED((N, HID), F32),
        pltpu.SemaphoreType.DMA,
        pltpu.SemaphoreType.DMA,
    ],
)(_segsum_body)


def _segsum(m, srcr, dstr, zeros):
    p = _segsum_call(m, srcr, dstr, zeros)
    return p.reshape(NC, N, HID)


def _dot(a, b):
    return jnp.dot(a, b, preferred_element_type=F32,
                   precision=lax.Precision.DEFAULT)


def _gru_math(x, h, gh, wih_t, bih):
    gi = _dot(x, wih_t) + bih
    r = jax.nn.sigmoid(gi[:, :HID] + gh[:, :HID])
    z = jax.nn.sigmoid(gi[:, HID:2 * HID] + gh[:, HID:2 * HID])
    n = jnp.tanh(gi[:, 2 * HID:] + r * gh[:, 2 * HID:])
    return (1.0 - z) * n + z * h


def _a1_body(x_ref, wih_ref, bih_ref, bhh_ref, w1t_ref, b1_ref,
             xi_ref, m1_ref):
    x = x_ref[...]
    gh = jnp.broadcast_to(bhh_ref[...], (BLK, 3 * HID))
    xi = _gru_math(x, jnp.zeros((BLK, HID), F32), gh, wih_ref[...],
                   bih_ref[...])
    xi_ref[...] = xi
    m1_ref[...] = _dot(xi, w1t_ref[...]) + b1_ref[...]


def _a_body(dt_ref, p2a_ref, p2b_ref, tprev_ref, h_ref, wih_ref, whh_ref,
            bih_ref, bhh_ref, w1t_ref, b1_ref,
            temp_ref, xii_ref, xi_ref, m1_ref):
    xii = p2a_ref[...] + p2b_ref[...]
    temp = tprev_ref[...] + dt_ref[0, 0] * xii
    h = h_ref[...]
    gh = _dot(h, whh_ref[...]) + bhh_ref[...]
    xi = _gru_math(temp, h, gh, wih_ref[...], bih_ref[...])
    temp_ref[...] = temp
    xii_ref[...] = xii
    xi_ref[...] = xi
    m1_ref[...] = _dot(xi, w1t_ref[...]) + b1_ref[...]


def _b_body(p1a_ref, p1b_ref, w2t_ref, b2_ref, m2_ref):
    h1 = jax.nn.relu(p1a_ref[...] + p1b_ref[...])
    m2_ref[...] = _dot(h1, w2t_ref[...]) + b2_ref[...]


def _c_body(p2a_ref, p2b_ref, xii_ref):
    xii_ref[...] = p2a_ref[...] + p2b_ref[...]


def _row_spec():
    return pl.BlockSpec((BLK, HID), lambda i: (i, 0))


def _full_spec(shape):
    return pl.BlockSpec(shape, lambda i: tuple(0 for _ in shape))


_GRID = N // BLK

_a1_call = pl.pallas_call(
    _a1_body,
    grid=(_GRID,),
    in_specs=[_row_spec(), _full_spec((HID, 3 * HID)), _full_spec((1, 3 * HID)),
              _full_spec((1, 3 * HID)), _full_spec((HID, HID)),
              _full_spec((1, HID))],
    out_specs=[_row_spec(), _row_spec()],
    out_shape=[jax.ShapeDtypeStruct((N, HID), F32),
               jax.ShapeDtypeStruct((N, HID), F32)],
)

_a_call = pl.pallas_call(
    _a_body,
    grid=(_GRID,),
    in_specs=[_full_spec((1, 1)), _row_spec(), _row_spec(), _row_spec(),
              _row_spec(), _full_spec((HID, 3 * HID)),
              _full_spec((HID, 3 * HID)), _full_spec((1, 3 * HID)),
              _full_spec((1, 3 * HID)), _full_spec((HID, HID)),
              _full_spec((1, HID))],
    out_specs=[_row_spec(), _row_spec(), _row_spec(), _row_spec()],
    out_shape=[jax.ShapeDtypeStruct((N, HID), F32)] * 4,
)

_b_call = pl.pallas_call(
    _b_body,
    grid=(_GRID,),
    in_specs=[_row_spec(), _row_spec(), _full_spec((HID, HID)),
              _full_spec((1, HID))],
    out_specs=[_row_spec()],
    out_shape=[jax.ShapeDtypeStruct((N, HID), F32)],
)

_c_call = pl.pallas_call(
    _c_body,
    grid=(_GRID,),
    in_specs=[_row_spec(), _row_spec()],
    out_specs=[_row_spec()],
    out_shape=[jax.ShapeDtypeStruct((N, HID), F32)],
)


def kernel(t, x0, edge_index, gru_w_ih, gru_w_hh, gru_b_ih, gru_b_hh,
           W1, b1, W2, b2):
    src = edge_index[0].reshape(NC, NS, NSTAGE, SCHUNK, CHUNK)
    dst = edge_index[1].reshape(NC, NS, NSTAGE, SCHUNK, CHUNK)
    zeros = jnp.zeros((ROWS_PER_S, HID), F32)

    wih_t = gru_w_ih.T
    whh_t = gru_w_hh.T
    bih = gru_b_ih.reshape(1, 3 * HID)
    bhh = gru_b_hh.reshape(1, 3 * HID)
    w1t = W1.T
    w2t = W2.T
    b1r = b1.reshape(1, HID)
    b2r = b2.reshape(1, HID)

    def gnn_partials(m1):
        p1 = _segsum(m1, src, dst, zeros)
        (m2,) = _b_call(p1[0], p1[1], w2t, b2r)
        p2 = _segsum(m2, src, dst, zeros)
        return p2

    xi, m1 = _a1_call(x0, wih_t, bih, bhh, w1t, b1r)
    h = xi
    temp = x0
    outs = [x0]
    douts = []
    for s in range(1, T):
        p2 = gnn_partials(m1)
        dt = (t[s] - t[s - 1]).reshape(1, 1)
        temp, xii, xi, m1 = _a_call(dt, p2[0], p2[1], temp, h,
                                    wih_t, whh_t, bih, bhh, w1t, b1r)
        h = xi
        outs.append(temp)
        douts.append(xii)

    p2 = gnn_partials(m1)
    (xii,) = _c_call(p2[0], p2[1])
    douts.append(xii)

    output = jnp.stack(outs, axis=0)
    doutput = jnp.stack(douts, axis=0)
    return jnp.concatenate([output, doutput], axis=-1)


# speedup vs baseline: 7.1268x; 1.0305x over previous
import functools

import jax
import jax.numpy as jnp
from jax import lax
from jax.experimental import pallas as pl
from jax.experimental.pallas import tpu as pltpu
from jax.experimental.pallas import tpu_sc as plsc

N = 10000
HID = 128
T = 8
E = 320000

NC = 2
NS = 16
EDGES_PER_W = E // (NC * NS)
CHUNK = 80
NCHUNK = EDGES_PER_W // CHUNK
NSTAGE = 5
SCHUNK = NCHUNK // NSTAGE
NPAIR = SCHUNK // 2
ROWS_PER_S = N // NS

BLK = 2000
F32 = jnp.float32


def _segsum_body(m_hbm, srcr, dstr, zeros_hbm, out_hbm,
                 src_v, dst_v, rows_a, rows_b, acc,
                 sem_ga, sem_gb, sem_sa, sem_sb):
    cid = lax.axis_index("c")
    sid = lax.axis_index("s")

    def gather(buf, sem, c):
        return pltpu.make_async_copy(m_hbm.at[src_v.at[c]], buf, sem)

    def scatter(buf, sem, c):
        return pltpu.make_async_copy(buf, acc.at[dst_v.at[c]], sem)

    pltpu.sync_copy(zeros_hbm, acc.at[pl.ds(sid * ROWS_PER_S, ROWS_PER_S)])
    plsc.subcore_barrier()

    def stage(k, carry):
        pltpu.sync_copy(srcr.at[cid, sid, k], src_v)
        pltpu.sync_copy(dstr.at[cid, sid, k], dst_v)
        gather(rows_a, sem_ga, 0).start()
        gather(rows_b, sem_gb, 1).start()

        def body(i, carry2):
            c0 = 2 * i
            c1 = c0 + 1
            gather(rows_a, sem_ga, c0).wait()
            scatter(rows_a, sem_sa, c0).start(add=True)
            gather(rows_b, sem_gb, c1).wait()
            scatter(rows_b, sem_sb, c1).start(add=True)
            scatter(rows_a, sem_sa, c0).wait()
            gather(rows_a, sem_ga, c0 + 2).start()
            scatter(rows_b, sem_sb, c1).wait()

            @pl.when(i < NPAIR - 1)
            def _():
                gather(rows_b, sem_gb, c1 + 2).start()

            return carry2

        lax.fori_loop(0, NPAIR, body, 0)

        c_last = SCHUNK - 1
        gather(rows_a, sem_ga, c_last).wait()
        pltpu.sync_copy(rows_a, acc.at[dst_v.at[c_last]], add=True)
        return carry

    lax.fori_loop(0, NSTAGE, stage, 0)

    plsc.subcore_barrier()
    pltpu.sync_copy(acc.at[pl.ds(sid * ROWS_PER_S, ROWS_PER_S)],
                    out_hbm.at[cid, sid])


_segsum_call = functools.partial(
    pl.kernel,
    mesh=plsc.VectorSubcoreMesh(core_axis_name="c", subcore_axis_name="s"),
    out_type=jax.ShapeDtypeStruct((NC, NS, ROWS_PER_S, HID), F32),
    scratch_types=[
        pltpu.VMEM((SCHUNK, CHUNK), jnp.int32),
        pltpu.VMEM((SCHUNK, CHUNK), jnp.int32),
        pltpu.VMEM((CHUNK, HID), F32),
        pltpu.VMEM((CHUNK, HID), F32),
        pltpu.VMEM_SHARED((N, HID), F32),
        pltpu.SemaphoreType.DMA,
        pltpu.SemaphoreType.DMA,
        pltpu.SemaphoreType.DMA,
        pltpu.SemaphoreType.DMA,
    ],
)(_segsum_body)


def _segsum(m, srcr, dstr, zeros):
    p = _segsum_call(m, srcr, dstr, zeros)
    return p.reshape(NC, N, HID)


def _dot(a, b):
    return jnp.dot(a, b, preferred_element_type=F32,
                   precision=lax.Precision.DEFAULT)


def _gru_math(x, h, gh, wih_t, bih):
    gi = _dot(x, wih_t) + bih
    r = jax.nn.sigmoid(gi[:, :HID] + gh[:, :HID])
    z = jax.nn.sigmoid(gi[:, HID:2 * HID] + gh[:, HID:2 * HID])
    n = jnp.tanh(gi[:, 2 * HID:] + r * gh[:, 2 * HID:])
    return (1.0 - z) * n + z * h


def _a1_body(x_ref, wih_ref, bih_ref, bhh_ref, w1t_ref, b1_ref,
             xi_ref, m1_ref):
    x = x_ref[...]
    gh = jnp.broadcast_to(bhh_ref[...], (BLK, 3 * HID))
    xi = _gru_math(x, jnp.zeros((BLK, HID), F32), gh, wih_ref[...],
                   bih_ref[...])
    xi_ref[...] = xi
    m1_ref[...] = _dot(xi, w1t_ref[...]) + b1_ref[...]


def _a_body(dt_ref, p2a_ref, p2b_ref, tprev_ref, h_ref, wih_ref, whh_ref,
            bih_ref, bhh_ref, w1t_ref, b1_ref,
            temp_ref, xii_ref, xi_ref, m1_ref):
    xii = p2a_ref[...] + p2b_ref[...]
    temp = tprev_ref[...] + dt_ref[0, 0] * xii
    h = h_ref[...]
    gh = _dot(h, whh_ref[...]) + bhh_ref[...]
    xi = _gru_math(temp, h, gh, wih_ref[...], bih_ref[...])
    temp_ref[...] = temp
    xii_ref[...] = xii
    xi_ref[...] = xi
    m1_ref[...] = _dot(xi, w1t_ref[...]) + b1_ref[...]


def _b_body(p1a_ref, p1b_ref, w2t_ref, b2_ref, m2_ref):
    h1 = jax.nn.relu(p1a_ref[...] + p1b_ref[...])
    m2_ref[...] = _dot(h1, w2t_ref[...]) + b2_ref[...]


def _c_body(p2a_ref, p2b_ref, xii_ref):
    xii_ref[...] = p2a_ref[...] + p2b_ref[...]


def _row_spec():
    return pl.BlockSpec((BLK, HID), lambda i: (i, 0))


def _full_spec(shape):
    return pl.BlockSpec(shape, lambda i: tuple(0 for _ in shape))


_GRID = N // BLK

_a1_call = pl.pallas_call(
    _a1_body,
    grid=(_GRID,),
    in_specs=[_row_spec(), _full_spec((HID, 3 * HID)), _full_spec((1, 3 * HID)),
              _full_spec((1, 3 * HID)), _full_spec((HID, HID)),
              _full_spec((1, HID))],
    out_specs=[_row_spec(), _row_spec()],
    out_shape=[jax.ShapeDtypeStruct((N, HID), F32),
               jax.ShapeDtypeStruct((N, HID), F32)],
)

_a_call = pl.pallas_call(
    _a_body,
    grid=(_GRID,),
    in_specs=[_full_spec((1, 1)), _row_spec(), _row_spec(), _row_spec(),
              _row_spec(), _full_spec((HID, 3 * HID)),
              _full_spec((HID, 3 * HID)), _full_spec((1, 3 * HID)),
              _full_spec((1, 3 * HID)), _full_spec((HID, HID)),
              _full_spec((1, HID))],
    out_specs=[_row_spec(), _row_spec(), _row_spec(), _row_spec()],
    out_shape=[jax.ShapeDtypeStruct((N, HID), F32)] * 4,
)

_b_call = pl.pallas_call(
    _b_body,
    grid=(_GRID,),
    in_specs=[_row_spec(), _row_spec(), _full_spec((HID, HID)),
              _full_spec((1, HID))],
    out_specs=[_row_spec()],
    out_shape=[jax.ShapeDtypeStruct((N, HID), F32)],
)

_c_call = pl.pallas_call(
    _c_body,
    grid=(_GRID,),
    in_specs=[_row_spec(), _row_spec()],
    out_specs=[_row_spec()],
    out_shape=[jax.ShapeDtypeStruct((N, HID), F32)],
)


def kernel(t, x0, edge_index, gru_w_ih, gru_w_hh, gru_b_ih, gru_b_hh,
           W1, b1, W2, b2):
    src = edge_index[0].reshape(NC, NS, NSTAGE, SCHUNK, CHUNK)
    dst = edge_index[1].reshape(NC, NS, NSTAGE, SCHUNK, CHUNK)
    zeros = jnp.zeros((ROWS_PER_S, HID), F32)

    wih_t = gru_w_ih.T
    whh_t = gru_w_hh.T
    bih = gru_b_ih.reshape(1, 3 * HID)
    bhh = gru_b_hh.reshape(1, 3 * HID)
    w1t = W1.T
    w2t = W2.T
    b1r = b1.reshape(1, HID)
    b2r = b2.reshape(1, HID)

    def gnn_partials(m1):
        p1 = _segsum(m1, src, dst, zeros)
        (m2,) = _b_call(p1[0], p1[1], w2t, b2r)
        p2 = _segsum(m2, src, dst, zeros)
        return p2

    xi, m1 = _a1_call(x0, wih_t, bih, bhh, w1t, b1r)
    h = xi
    temp = x0
    outs = [x0]
    douts = []
    for s in range(1, T):
        p2 = gnn_partials(m1)
        dt = (t[s] - t[s - 1]).reshape(1, 1)
        temp, xii, xi, m1 = _a_call(dt, p2[0], p2[1], temp, h,
                                    wih_t, whh_t, bih, bhh, w1t, b1r)
        h = xi
        outs.append(temp)
        douts.append(xii)

    p2 = gnn_partials(m1)
    (xii,) = _c_call(p2[0], p2[1])
    douts.append(xii)

    output = jnp.stack(outs, axis=0)
    doutput = jnp.stack(douts, axis=0)
    return jnp.concatenate([output, doutput], axis=-1)


# speedup vs baseline: 8.8526x; 1.2422x over previous
import functools

import jax
import jax.numpy as jnp
from jax import lax
from jax.experimental import pallas as pl
from jax.experimental.pallas import tpu as pltpu
from jax.experimental.pallas import tpu_sc as plsc

N = 10000
HID = 128
T = 8
E = 320000

NC = 2
NS = 16
EDGES_PER_W = E // (NC * NS)
CHUNK = 40
NCHUNK = EDGES_PER_W // CHUNK
NSTAGE = 5
SCHUNK = NCHUNK // NSTAGE
NQUAD = (SCHUNK - 2) // 4
ROWS_PER_S = N // NS

BLK = 2000
F32 = jnp.float32


def _segsum_body(m_hbm, srcr, dstr, zeros_hbm, out_hbm,
                 src_v, dst_v, rows_a, rows_b, rows_c, rows_d, acc,
                 sem_ga, sem_gb, sem_gc, sem_gd,
                 sem_sa, sem_sb, sem_sc, sem_sd):
    cid = lax.axis_index("c")
    sid = lax.axis_index("s")

    def gather(buf, sem, c):
        return pltpu.make_async_copy(m_hbm.at[src_v.at[c]], buf, sem)

    def scatter(buf, sem, c):
        return pltpu.make_async_copy(buf, acc.at[dst_v.at[c]], sem)

    pltpu.sync_copy(zeros_hbm, acc.at[pl.ds(sid * ROWS_PER_S, ROWS_PER_S)])
    plsc.subcore_barrier()

    def stage(k, carry):
        pltpu.sync_copy(srcr.at[cid, sid, k], src_v)
        pltpu.sync_copy(dstr.at[cid, sid, k], dst_v)
        gather(rows_a, sem_ga, 0).start()
        gather(rows_b, sem_gb, 1).start()
        gather(rows_c, sem_gc, 2).start()
        gather(rows_d, sem_gd, 3).start()

        def body(i, carry2):
            q0 = 4 * i
            gather(rows_a, sem_ga, q0).wait()
            scatter(rows_a, sem_sa, q0).start(add=True)
            gather(rows_b, sem_gb, q0 + 1).wait()
            scatter(rows_b, sem_sb, q0 + 1).start(add=True)
            scatter(rows_a, sem_sa, q0).wait()
            gather(rows_a, sem_ga, q0 + 4).start()
            gather(rows_c, sem_gc, q0 + 2).wait()
            scatter(rows_c, sem_sc, q0 + 2).start(add=True)
            scatter(rows_b, sem_sb, q0 + 1).wait()
            gather(rows_b, sem_gb, q0 + 5).start()
            gather(rows_d, sem_gd, q0 + 3).wait()
            scatter(rows_d, sem_sd, q0 + 3).start(add=True)
            scatter(rows_c, sem_sc, q0 + 2).wait()

            @pl.when(i < NQUAD - 1)
            def _():
                gather(rows_c, sem_gc, q0 + 6).start()

            scatter(rows_d, sem_sd, q0 + 3).wait()

            @pl.when(i < NQUAD - 1)
            def _():
                gather(rows_d, sem_gd, q0 + 7).start()

            return carry2

        lax.fori_loop(0, NQUAD, body, 0)

        gather(rows_a, sem_ga, SCHUNK - 2).wait()
        scatter(rows_a, sem_sa, SCHUNK - 2).start(add=True)
        gather(rows_b, sem_gb, SCHUNK - 1).wait()
        scatter(rows_b, sem_sb, SCHUNK - 1).start(add=True)
        scatter(rows_a, sem_sa, SCHUNK - 2).wait()
        scatter(rows_b, sem_sb, SCHUNK - 1).wait()
        return carry

    lax.fori_loop(0, NSTAGE, stage, 0)

    plsc.subcore_barrier()
    pltpu.sync_copy(acc.at[pl.ds(sid * ROWS_PER_S, ROWS_PER_S)],
                    out_hbm.at[cid, sid])


_segsum_call = functools.partial(
    pl.kernel,
    mesh=plsc.VectorSubcoreMesh(core_axis_name="c", subcore_axis_name="s"),
    out_type=jax.ShapeDtypeStruct((NC, NS, ROWS_PER_S, HID), F32),
    scratch_types=[
        pltpu.VMEM((SCHUNK, CHUNK), jnp.int32),
        pltpu.VMEM((SCHUNK, CHUNK), jnp.int32),
        pltpu.VMEM((CHUNK, HID), F32),
        pltpu.VMEM((CHUNK, HID), F32),
        pltpu.VMEM((CHUNK, HID), F32),
        pltpu.VMEM((CHUNK, HID), F32),
        pltpu.VMEM_SHARED((N, HID), F32),
        pltpu.SemaphoreType.DMA,
        pltpu.SemaphoreType.DMA,
        pltpu.SemaphoreType.DMA,
        pltpu.SemaphoreType.DMA,
        pltpu.SemaphoreType.DMA,
        pltpu.SemaphoreType.DMA,
        pltpu.SemaphoreType.DMA,
        pltpu.SemaphoreType.DMA,
    ],
)(_segsum_body)


def _segsum(m, srcr, dstr, zeros):
    p = _segsum_call(m, srcr, dstr, zeros)
    return p.reshape(NC, N, HID)


def _dot(a, b):
    return jnp.dot(a, b, preferred_element_type=F32,
                   precision=lax.Precision.DEFAULT)


def _gru_math(x, h, gh, wih_t, bih):
    gi = _dot(x, wih_t) + bih
    r = jax.nn.sigmoid(gi[:, :HID] + gh[:, :HID])
    z = jax.nn.sigmoid(gi[:, HID:2 * HID] + gh[:, HID:2 * HID])
    n = jnp.tanh(gi[:, 2 * HID:] + r * gh[:, 2 * HID:])
    return (1.0 - z) * n + z * h


def _a1_body(x_ref, wih_ref, bih_ref, bhh_ref, w1t_ref, b1_ref,
             xi_ref, m1_ref):
    x = x_ref[...]
    gh = jnp.broadcast_to(bhh_ref[...], (BLK, 3 * HID))
    xi = _gru_math(x, jnp.zeros((BLK, HID), F32), gh, wih_ref[...],
                   bih_ref[...])
    xi_ref[...] = xi
    m1_ref[...] = _dot(xi, w1t_ref[...]) + b1_ref[...]


def _a_body(dt_ref, p2a_ref, p2b_ref, tprev_ref, h_ref, wih_ref, whh_ref,
            bih_ref, bhh_ref, w1t_ref, b1_ref,
            temp_ref, xii_ref, xi_ref, m1_ref):
    xii = p2a_ref[...] + p2b_ref[...]
    temp = tprev_ref[...] + dt_ref[0, 0] * xii
    h = h_ref[...]
    gh = _dot(h, whh_ref[...]) + bhh_ref[...]
    xi = _gru_math(temp, h, gh, wih_ref[...], bih_ref[...])
    temp_ref[...] = temp
    xii_ref[...] = xii
    xi_ref[...] = xi
    m1_ref[...] = _dot(xi, w1t_ref[...]) + b1_ref[...]


def _b_body(p1a_ref, p1b_ref, w2t_ref, b2_ref, m2_ref):
    h1 = jax.nn.relu(p1a_ref[...] + p1b_ref[...])
    m2_ref[...] = _dot(h1, w2t_ref[...]) + b2_ref[...]


def _c_body(p2a_ref, p2b_ref, xii_ref):
    xii_ref[...] = p2a_ref[...] + p2b_ref[...]


def _row_spec():
    return pl.BlockSpec((BLK, HID), lambda i: (i, 0))


def _full_spec(shape):
    return pl.BlockSpec(shape, lambda i: tuple(0 for _ in shape))


_GRID = N // BLK

_a1_call = pl.pallas_call(
    _a1_body,
    grid=(_GRID,),
    in_specs=[_row_spec(), _full_spec((HID, 3 * HID)), _full_spec((1, 3 * HID)),
              _full_spec((1, 3 * HID)), _full_spec((HID, HID)),
              _full_spec((1, HID))],
    out_specs=[_row_spec(), _row_spec()],
    out_shape=[jax.ShapeDtypeStruct((N, HID), F32),
               jax.ShapeDtypeStruct((N, HID), F32)],
)

_a_call = pl.pallas_call(
    _a_body,
    grid=(_GRID,),
    in_specs=[_full_spec((1, 1)), _row_spec(), _row_spec(), _row_spec(),
              _row_spec(), _full_spec((HID, 3 * HID)),
              _full_spec((HID, 3 * HID)), _full_spec((1, 3 * HID)),
              _full_spec((1, 3 * HID)), _full_spec((HID, HID)),
              _full_spec((1, HID))],
    out_specs=[_row_spec(), _row_spec(), _row_spec(), _row_spec()],
    out_shape=[jax.ShapeDtypeStruct((N, HID), F32)] * 4,
)

_b_call = pl.pallas_call(
    _b_body,
    grid=(_GRID,),
    in_specs=[_row_spec(), _row_spec(), _full_spec((HID, HID)),
              _full_spec((1, HID))],
    out_specs=[_row_spec()],
    out_shape=[jax.ShapeDtypeStruct((N, HID), F32)],
)

_c_call = pl.pallas_call(
    _c_body,
    grid=(_GRID,),
    in_specs=[_row_spec(), _row_spec()],
    out_specs=[_row_spec()],
    out_shape=[jax.ShapeDtypeStruct((N, HID), F32)],
)


def kernel(t, x0, edge_index, gru_w_ih, gru_w_hh, gru_b_ih, gru_b_hh,
           W1, b1, W2, b2):
    src = edge_index[0].reshape(NC, NS, NSTAGE, SCHUNK, CHUNK)
    dst = edge_index[1].reshape(NC, NS, NSTAGE, SCHUNK, CHUNK)
    zeros = jnp.zeros((ROWS_PER_S, HID), F32)

    wih_t = gru_w_ih.T
    whh_t = gru_w_hh.T
    bih = gru_b_ih.reshape(1, 3 * HID)
    bhh = gru_b_hh.reshape(1, 3 * HID)
    w1t = W1.T
    w2t = W2.T
    b1r = b1.reshape(1, HID)
    b2r = b2.reshape(1, HID)

    def gnn_partials(m1):
        p1 = _segsum(m1, src, dst, zeros)
        (m2,) = _b_call(p1[0], p1[1], w2t, b2r)
        p2 = _segsum(m2, src, dst, zeros)
        return p2

    xi, m1 = _a1_call(x0, wih_t, bih, bhh, w1t, b1r)
    h = xi
    temp = x0
    outs = [x0]
    douts = []
    for s in range(1, T):
        p2 = gnn_partials(m1)
        dt = (t[s] - t[s - 1]).reshape(1, 1)
        temp, xii, xi, m1 = _a_call(dt, p2[0], p2[1], temp, h,
                                    wih_t, whh_t, bih, bhh, w1t, b1r)
        h = xi
        outs.append(temp)
        douts.append(xii)

    p2 = gnn_partials(m1)
    (xii,) = _c_call(p2[0], p2[1])
    douts.append(xii)

    output = jnp.stack(outs, axis=0)
    doutput = jnp.stack(douts, axis=0)
    return jnp.concatenate([output, doutput], axis=-1)


# speedup vs baseline: 9.0978x; 1.0277x over previous
import functools

import jax
import jax.numpy as jnp
from jax import lax
from jax.experimental import pallas as pl
from jax.experimental.pallas import tpu as pltpu
from jax.experimental.pallas import tpu_sc as plsc

N = 10000
HID = 128
T = 8
E = 320000

NC = 2
NS = 16
EDGES_PER_W = E // (NC * NS)
CHUNK = 40
NCHUNK = EDGES_PER_W // CHUNK
NSTAGE = 5
SCHUNK = NCHUNK // NSTAGE
NQUAD = (SCHUNK - 2) // 4
ROWS_PER_S = N // NS

BLK = 2000
F32 = jnp.float32


def _segsum_body(m_hbm, srcr, dstr, zeros_hbm, out_hbm,
                 src_a, dst_a, src_b, dst_b,
                 rows_a, rows_b, rows_c, rows_d, acc,
                 sem_ix, sem_ga, sem_gb, sem_gc, sem_gd,
                 sem_sa, sem_sb, sem_sc, sem_sd):
    cid = lax.axis_index("c")
    sid = lax.axis_index("s")

    pltpu.sync_copy(zeros_hbm, acc.at[pl.ds(sid * ROWS_PER_S, ROWS_PER_S)])

    pltpu.sync_copy(srcr.at[cid, sid, 0], src_a)
    pltpu.sync_copy(dstr.at[cid, sid, 0], dst_a)
    plsc.subcore_barrier()

    for k in range(NSTAGE):
        src_v, dst_v = (src_a, dst_a) if k % 2 == 0 else (src_b, dst_b)
        nsrc_v, ndst_v = (src_b, dst_b) if k % 2 == 0 else (src_a, dst_a)

        def gather(buf, sem, c, src_v=src_v):
            return pltpu.make_async_copy(m_hbm.at[src_v.at[c]], buf, sem)

        def scatter(buf, sem, c, dst_v=dst_v):
            return pltpu.make_async_copy(buf, acc.at[dst_v.at[c]], sem)

        gather(rows_a, sem_ga, 0).start()
        gather(rows_b, sem_gb, 1).start()
        gather(rows_c, sem_gc, 2).start()
        gather(rows_d, sem_gd, 3).start()

        if k < NSTAGE - 1:
            pltpu.make_async_copy(srcr.at[cid, sid, k + 1], nsrc_v,
                                  sem_ix).start()
            pltpu.make_async_copy(dstr.at[cid, sid, k + 1], ndst_v,
                                  sem_ix).start()

        def body(i, carry2, gather=gather, scatter=scatter):
            q0 = 4 * i
            gather(rows_a, sem_ga, q0).wait()
            scatter(rows_a, sem_sa, q0).start(add=True)
            gather(rows_b, sem_gb, q0 + 1).wait()
            scatter(rows_b, sem_sb, q0 + 1).start(add=True)
            scatter(rows_a, sem_sa, q0).wait()
            gather(rows_a, sem_ga, q0 + 4).start()
            gather(rows_c, sem_gc, q0 + 2).wait()
            scatter(rows_c, sem_sc, q0 + 2).start(add=True)
            scatter(rows_b, sem_sb, q0 + 1).wait()
            gather(rows_b, sem_gb, q0 + 5).start()
            gather(rows_d, sem_gd, q0 + 3).wait()
            scatter(rows_d, sem_sd, q0 + 3).start(add=True)
            scatter(rows_c, sem_sc, q0 + 2).wait()

            @pl.when(i < NQUAD - 1)
            def _():
                gather(rows_c, sem_gc, q0 + 6).start()

            scatter(rows_d, sem_sd, q0 + 3).wait()

            @pl.when(i < NQUAD - 1)
            def _():
                gather(rows_d, sem_gd, q0 + 7).start()

            return carry2

        lax.fori_loop(0, NQUAD, body, 0)

        gather(rows_a, sem_ga, SCHUNK - 2).wait()
        scatter(rows_a, sem_sa, SCHUNK - 2).start(add=True)
        gather(rows_b, sem_gb, SCHUNK - 1).wait()
        scatter(rows_b, sem_sb, SCHUNK - 1).start(add=True)
        scatter(rows_a, sem_sa, SCHUNK - 2).wait()
        scatter(rows_b, sem_sb, SCHUNK - 1).wait()

        if k < NSTAGE - 1:
            pltpu.make_async_copy(srcr.at[cid, sid, k + 1], nsrc_v,
                                  sem_ix).wait()
            pltpu.make_async_copy(dstr.at[cid, sid, k + 1], ndst_v,
                                  sem_ix).wait()

    plsc.subcore_barrier()
    pltpu.sync_copy(acc.at[pl.ds(sid * ROWS_PER_S, ROWS_PER_S)],
                    out_hbm.at[cid, sid])


_segsum_call = functools.partial(
    pl.kernel,
    mesh=plsc.VectorSubcoreMesh(core_axis_name="c", subcore_axis_name="s"),
    out_type=jax.ShapeDtypeStruct((NC, NS, ROWS_PER_S, HID), F32),
    scratch_types=[
        pltpu.VMEM((SCHUNK, CHUNK), jnp.int32),
        pltpu.VMEM((SCHUNK, CHUNK), jnp.int32),
        pltpu.VMEM((SCHUNK, CHUNK), jnp.int32),
        pltpu.VMEM((SCHUNK, CHUNK), jnp.int32),
        pltpu.VMEM((CHUNK, HID), F32),
        pltpu.VMEM((CHUNK, HID), F32),
        pltpu.VMEM((CHUNK, HID), F32),
        pltpu.VMEM((CHUNK, HID), F32),
        pltpu.VMEM_SHARED((N, HID), F32),
        pltpu.SemaphoreType.DMA,
        pltpu.SemaphoreType.DMA,
        pltpu.SemaphoreType.DMA,
        pltpu.SemaphoreType.DMA,
        pltpu.SemaphoreType.DMA,
        pltpu.SemaphoreType.DMA,
        pltpu.SemaphoreType.DMA,
        pltpu.SemaphoreType.DMA,
        pltpu.SemaphoreType.DMA,
    ],
)(_segsum_body)


def _segsum(m, srcr, dstr, zeros):
    p = _segsum_call(m, srcr, dstr, zeros)
    return p.reshape(NC, N, HID)


def _dot(a, b):
    return jnp.dot(a, b, preferred_element_type=F32,
                   precision=lax.Precision.DEFAULT)


def _gru_math(x, h, gh, wih_t, bih):
    gi = _dot(x, wih_t) + bih
    r = jax.nn.sigmoid(gi[:, :HID] + gh[:, :HID])
    z = jax.nn.sigmoid(gi[:, HID:2 * HID] + gh[:, HID:2 * HID])
    n = jnp.tanh(gi[:, 2 * HID:] + r * gh[:, 2 * HID:])
    return (1.0 - z) * n + z * h


def _a1_body(x_ref, wih_ref, bih_ref, bhh_ref, w1t_ref, b1_ref,
             xi_ref, m1_ref):
    x = x_ref[...]
    gh = jnp.broadcast_to(bhh_ref[...], (BLK, 3 * HID))
    xi = _gru_math(x, jnp.zeros((BLK, HID), F32), gh, wih_ref[...],
                   bih_ref[...])
    xi_ref[...] = xi
    m1_ref[...] = _dot(xi, w1t_ref[...]) + b1_ref[...]


def _a_body(dt_ref, p2a_ref, p2b_ref, tprev_ref, h_ref, wih_ref, whh_ref,
            bih_ref, bhh_ref, w1t_ref, b1_ref,
            temp_ref, xii_ref, xi_ref, m1_ref):
    xii = p2a_ref[...] + p2b_ref[...]
    temp = tprev_ref[...] + dt_ref[0, 0] * xii
    h = h_ref[...]
    gh = _dot(h, whh_ref[...]) + bhh_ref[...]
    xi = _gru_math(temp, h, gh, wih_ref[...], bih_ref[...])
    temp_ref[...] = temp
    xii_ref[...] = xii
    xi_ref[...] = xi
    m1_ref[...] = _dot(xi, w1t_ref[...]) + b1_ref[...]


def _b_body(p1a_ref, p1b_ref, w2t_ref, b2_ref, m2_ref):
    h1 = jax.nn.relu(p1a_ref[...] + p1b_ref[...])
    m2_ref[...] = _dot(h1, w2t_ref[...]) + b2_ref[...]


def _c_body(p2a_ref, p2b_ref, xii_ref):
    xii_ref[...] = p2a_ref[...] + p2b_ref[...]


def _row_spec():
    return pl.BlockSpec((BLK, HID), lambda i: (i, 0))


def _full_spec(shape):
    return pl.BlockSpec(shape, lambda i: tuple(0 for _ in shape))


_GRID = N // BLK

_a1_call = pl.pallas_call(
    _a1_body,
    grid=(_GRID,),
    in_specs=[_row_spec(), _full_spec((HID, 3 * HID)), _full_spec((1, 3 * HID)),
              _full_spec((1, 3 * HID)), _full_spec((HID, HID)),
              _full_spec((1, HID))],
    out_specs=[_row_spec(), _row_spec()],
    out_shape=[jax.ShapeDtypeStruct((N, HID), F32),
               jax.ShapeDtypeStruct((N, HID), F32)],
)

_a_call = pl.pallas_call(
    _a_body,
    grid=(_GRID,),
    in_specs=[_full_spec((1, 1)), _row_spec(), _row_spec(), _row_spec(),
              _row_spec(), _full_spec((HID, 3 * HID)),
              _full_spec((HID, 3 * HID)), _full_spec((1, 3 * HID)),
              _full_spec((1, 3 * HID)), _full_spec((HID, HID)),
              _full_spec((1, HID))],
    out_specs=[_row_spec(), _row_spec(), _row_spec(), _row_spec()],
    out_shape=[jax.ShapeDtypeStruct((N, HID), F32)] * 4,
)

_b_call = pl.pallas_call(
    _b_body,
    grid=(_GRID,),
    in_specs=[_row_spec(), _row_spec(), _full_spec((HID, HID)),
              _full_spec((1, HID))],
    out_specs=[_row_spec()],
    out_shape=[jax.ShapeDtypeStruct((N, HID), F32)],
)

_c_call = pl.pallas_call(
    _c_body,
    grid=(_GRID,),
    in_specs=[_row_spec(), _row_spec()],
    out_specs=[_row_spec()],
    out_shape=[jax.ShapeDtypeStruct((N, HID), F32)],
)


def kernel(t, x0, edge_index, gru_w_ih, gru_w_hh, gru_b_ih, gru_b_hh,
           W1, b1, W2, b2):
    src = edge_index[0].reshape(NC, NS, NSTAGE, SCHUNK, CHUNK)
    dst = edge_index[1].reshape(NC, NS, NSTAGE, SCHUNK, CHUNK)
    zeros = jnp.zeros((ROWS_PER_S, HID), F32)

    wih_t = gru_w_ih.T
    whh_t = gru_w_hh.T
    bih = gru_b_ih.reshape(1, 3 * HID)
    bhh = gru_b_hh.reshape(1, 3 * HID)
    w1t = W1.T
    w2t = W2.T
    b1r = b1.reshape(1, HID)
    b2r = b2.reshape(1, HID)

    def gnn_partials(m1):
        p1 = _segsum(m1, src, dst, zeros)
        (m2,) = _b_call(p1[0], p1[1], w2t, b2r)
        p2 = _segsum(m2, src, dst, zeros)
        return p2

    xi, m1 = _a1_call(x0, wih_t, bih, bhh, w1t, b1r)
    h = xi
    temp = x0
    outs = [x0]
    douts = []
    for s in range(1, T):
        p2 = gnn_partials(m1)
        dt = (t[s] - t[s - 1]).reshape(1, 1)
        temp, xii, xi, m1 = _a_call(dt, p2[0], p2[1], temp, h,
                                    wih_t, whh_t, bih, bhh, w1t, b1r)
        h = xi
        outs.append(temp)
        douts.append(xii)

    p2 = gnn_partials(m1)
    (xii,) = _c_call(p2[0], p2[1])
    douts.append(xii)

    output = jnp.stack(outs, axis=0)
    doutput = jnp.stack(douts, axis=0)
    return jnp.concatenate([output, doutput], axis=-1)


# speedup vs baseline: 9.1731x; 1.0083x over previous
import functools

import jax
import jax.numpy as jnp
from jax import lax
from jax.experimental import pallas as pl
from jax.experimental.pallas import tpu as pltpu
from jax.experimental.pallas import tpu_sc as plsc

N = 10000
HID = 128
T = 8
E = 320000

NC = 2
NS = 16
EDGES_PER_W = E // (NC * NS)
CHUNK = 40
NCHUNK = EDGES_PER_W // CHUNK
NSTAGE = 5
SCHUNK = NCHUNK // NSTAGE
NQUAD = (SCHUNK - 2) // 4
ROWS_PER_S = N // NS

BLK = 2000
F32 = jnp.float32


def _segsum_body(m_hbm, srcr, dstr, zeros_hbm, out_hbm,
                 src_a, dst_a, src_b, dst_b,
                 rows_a, rows_b, rows_c, rows_d, acc,
                 sem_ix, sem_ga, sem_gb, sem_gc, sem_gd,
                 sem_sa, sem_sb, sem_sc, sem_sd):
    cid = lax.axis_index("c")
    sid = lax.axis_index("s")

    pltpu.sync_copy(srcr.at[cid, sid, 0], src_a)
    pltpu.sync_copy(dstr.at[cid, sid, 0], dst_a)
    pltpu.make_async_copy(m_hbm.at[src_a.at[0]], rows_a, sem_ga).start()
    pltpu.make_async_copy(m_hbm.at[src_a.at[1]], rows_b, sem_gb).start()
    pltpu.make_async_copy(m_hbm.at[src_a.at[2]], rows_c, sem_gc).start()
    pltpu.make_async_copy(m_hbm.at[src_a.at[3]], rows_d, sem_gd).start()
    pltpu.sync_copy(zeros_hbm, acc.at[pl.ds(sid * ROWS_PER_S, ROWS_PER_S)])
    plsc.subcore_barrier()

    for k in range(NSTAGE):
        src_v, dst_v = (src_a, dst_a) if k % 2 == 0 else (src_b, dst_b)
        nsrc_v, ndst_v = (src_b, dst_b) if k % 2 == 0 else (src_a, dst_a)

        def gather(buf, sem, c, src_v=src_v):
            return pltpu.make_async_copy(m_hbm.at[src_v.at[c]], buf, sem)

        def scatter(buf, sem, c, dst_v=dst_v):
            return pltpu.make_async_copy(buf, acc.at[dst_v.at[c]], sem)

        if k > 0:
            gather(rows_a, sem_ga, 0).start()
            gather(rows_b, sem_gb, 1).start()
            gather(rows_c, sem_gc, 2).start()
            gather(rows_d, sem_gd, 3).start()

        if k < NSTAGE - 1:
            pltpu.make_async_copy(srcr.at[cid, sid, k + 1], nsrc_v,
                                  sem_ix).start()
            pltpu.make_async_copy(dstr.at[cid, sid, k + 1], ndst_v,
                                  sem_ix).start()

        def body(i, carry2, gather=gather, scatter=scatter):
            q0 = 4 * i
            gather(rows_a, sem_ga, q0).wait()
            scatter(rows_a, sem_sa, q0).start(add=True)
            gather(rows_b, sem_gb, q0 + 1).wait()
            scatter(rows_b, sem_sb, q0 + 1).start(add=True)
            scatter(rows_a, sem_sa, q0).wait()
            gather(rows_a, sem_ga, q0 + 4).start()
            gather(rows_c, sem_gc, q0 + 2).wait()
            scatter(rows_c, sem_sc, q0 + 2).start(add=True)
            scatter(rows_b, sem_sb, q0 + 1).wait()
            gather(rows_b, sem_gb, q0 + 5).start()
            gather(rows_d, sem_gd, q0 + 3).wait()
            scatter(rows_d, sem_sd, q0 + 3).start(add=True)
            scatter(rows_c, sem_sc, q0 + 2).wait()

            @pl.when(i < NQUAD - 1)
            def _():
                gather(rows_c, sem_gc, q0 + 6).start()

            scatter(rows_d, sem_sd, q0 + 3).wait()

            @pl.when(i < NQUAD - 1)
            def _():
                gather(rows_d, sem_gd, q0 + 7).start()

            return carry2

        lax.fori_loop(0, NQUAD, body, 0)

        gather(rows_a, sem_ga, SCHUNK - 2).wait()
        scatter(rows_a, sem_sa, SCHUNK - 2).start(add=True)
        gather(rows_b, sem_gb, SCHUNK - 1).wait()
        scatter(rows_b, sem_sb, SCHUNK - 1).start(add=True)
        scatter(rows_a, sem_sa, SCHUNK - 2).wait()
        scatter(rows_b, sem_sb, SCHUNK - 1).wait()

        if k < NSTAGE - 1:
            pltpu.make_async_copy(srcr.at[cid, sid, k + 1], nsrc_v,
                                  sem_ix).wait()
            pltpu.make_async_copy(dstr.at[cid, sid, k + 1], ndst_v,
                                  sem_ix).wait()

    plsc.subcore_barrier()
    pltpu.sync_copy(acc.at[pl.ds(sid * ROWS_PER_S, ROWS_PER_S)],
                    out_hbm.at[cid, sid])


_segsum_call = functools.partial(
    pl.kernel,
    mesh=plsc.VectorSubcoreMesh(core_axis_name="c", subcore_axis_name="s"),
    out_type=jax.ShapeDtypeStruct((NC, NS, ROWS_PER_S, HID), F32),
    scratch_types=[
        pltpu.VMEM((SCHUNK, CHUNK), jnp.int32),
        pltpu.VMEM((SCHUNK, CHUNK), jnp.int32),
        pltpu.VMEM((SCHUNK, CHUNK), jnp.int32),
        pltpu.VMEM((SCHUNK, CHUNK), jnp.int32),
        pltpu.VMEM((CHUNK, HID), F32),
        pltpu.VMEM((CHUNK, HID), F32),
        pltpu.VMEM((CHUNK, HID), F32),
        pltpu.VMEM((CHUNK, HID), F32),
        pltpu.VMEM_SHARED((N, HID), F32),
        pltpu.SemaphoreType.DMA,
        pltpu.SemaphoreType.DMA,
        pltpu.SemaphoreType.DMA,
        pltpu.SemaphoreType.DMA,
        pltpu.SemaphoreType.DMA,
        pltpu.SemaphoreType.DMA,
        pltpu.SemaphoreType.DMA,
        pltpu.SemaphoreType.DMA,
        pltpu.SemaphoreType.DMA,
    ],
)(_segsum_body)


def _segsum(m, srcr, dstr, zeros):
    p = _segsum_call(m, srcr, dstr, zeros)
    return p.reshape(NC, N, HID)


def _dot(a, b):
    return jnp.dot(a, b, preferred_element_type=F32,
                   precision=lax.Precision.DEFAULT)


def _gru_math(x, h, gh, wih_t, bih):
    gi = _dot(x, wih_t) + bih
    r = jax.nn.sigmoid(gi[:, :HID] + gh[:, :HID])
    z = jax.nn.sigmoid(gi[:, HID:2 * HID] + gh[:, HID:2 * HID])
    n = jnp.tanh(gi[:, 2 * HID:] + r * gh[:, 2 * HID:])
    return (1.0 - z) * n + z * h


def _a1_body(x_ref, wih_ref, bih_ref, bhh_ref, w1t_ref, b1_ref,
             xi_ref, m1_ref):
    x = x_ref[...]
    gh = jnp.broadcast_to(bhh_ref[...], (BLK, 3 * HID))
    xi = _gru_math(x, jnp.zeros((BLK, HID), F32), gh, wih_ref[...],
                   bih_ref[...])
    xi_ref[...] = xi
    m1_ref[...] = _dot(xi, w1t_ref[...]) + b1_ref[...]


def _a_body(dt_ref, p2a_ref, p2b_ref, tprev_ref, h_ref, wih_ref, whh_ref,
            bih_ref, bhh_ref, w1t_ref, b1_ref,
            temp_ref, xii_ref, xi_ref, m1_ref):
    xii = p2a_ref[...] + p2b_ref[...]
    temp = tprev_ref[...] + dt_ref[0, 0] * xii
    h = h_ref[...]
    gh = _dot(h, whh_ref[...]) + bhh_ref[...]
    xi = _gru_math(temp, h, gh, wih_ref[...], bih_ref[...])
    temp_ref[...] = temp
    xii_ref[...] = xii
    xi_ref[...] = xi
    m1_ref[...] = _dot(xi, w1t_ref[...]) + b1_ref[...]


def _b_body(p1a_ref, p1b_ref, w2t_ref, b2_ref, m2_ref):
    h1 = jax.nn.relu(p1a_ref[...] + p1b_ref[...])
    m2_ref[...] = _dot(h1, w2t_ref[...]) + b2_ref[...]


def _c_body(p2a_ref, p2b_ref, xii_ref):
    xii_ref[...] = p2a_ref[...] + p2b_ref[...]


def _row_spec():
    return pl.BlockSpec((BLK, HID), lambda i: (i, 0))


def _full_spec(shape):
    return pl.BlockSpec(shape, lambda i: tuple(0 for _ in shape))


_GRID = N // BLK

_a1_call = pl.pallas_call(
    _a1_body,
    grid=(_GRID,),
    in_specs=[_row_spec(), _full_spec((HID, 3 * HID)), _full_spec((1, 3 * HID)),
              _full_spec((1, 3 * HID)), _full_spec((HID, HID)),
              _full_spec((1, HID))],
    out_specs=[_row_spec(), _row_spec()],
    out_shape=[jax.ShapeDtypeStruct((N, HID), F32),
               jax.ShapeDtypeStruct((N, HID), F32)],
)

_a_call = pl.pallas_call(
    _a_body,
    grid=(_GRID,),
    in_specs=[_full_spec((1, 1)), _row_spec(), _row_spec(), _row_spec(),
              _row_spec(), _full_spec((HID, 3 * HID)),
              _full_spec((HID, 3 * HID)), _full_spec((1, 3 * HID)),
              _full_spec((1, 3 * HID)), _full_spec((HID, HID)),
              _full_spec((1, HID))],
    out_specs=[_row_spec(), _row_spec(), _row_spec(), _row_spec()],
    out_shape=[jax.ShapeDtypeStruct((N, HID), F32)] * 4,
)

_b_call = pl.pallas_call(
    _b_body,
    grid=(_GRID,),
    in_specs=[_row_spec(), _row_spec(), _full_spec((HID, HID)),
              _full_spec((1, HID))],
    out_specs=[_row_spec()],
    out_shape=[jax.ShapeDtypeStruct((N, HID), F32)],
)

_c_call = pl.pallas_call(
    _c_body,
    grid=(_GRID,),
    in_specs=[_row_spec(), _row_spec()],
    out_specs=[_row_spec()],
    out_shape=[jax.ShapeDtypeStruct((N, HID), F32)],
)


def kernel(t, x0, edge_index, gru_w_ih, gru_w_hh, gru_b_ih, gru_b_hh,
           W1, b1, W2, b2):
    src = edge_index[0].reshape(NC, NS, NSTAGE, SCHUNK, CHUNK)
    dst = edge_index[1].reshape(NC, NS, NSTAGE, SCHUNK, CHUNK)
    zeros = jnp.zeros((ROWS_PER_S, HID), F32)

    wih_t = gru_w_ih.T
    whh_t = gru_w_hh.T
    bih = gru_b_ih.reshape(1, 3 * HID)
    bhh = gru_b_hh.reshape(1, 3 * HID)
    w1t = W1.T
    w2t = W2.T
    b1r = b1.reshape(1, HID)
    b2r = b2.reshape(1, HID)

    def gnn_partials(m1):
        p1 = _segsum(m1, src, dst, zeros)
        (m2,) = _b_call(p1[0], p1[1], w2t, b2r)
        p2 = _segsum(m2, src, dst, zeros)
        return p2

    xi, m1 = _a1_call(x0, wih_t, bih, bhh, w1t, b1r)
    h = xi
    temp = x0
    outs = [x0]
    douts = []
    for s in range(1, T):
        p2 = gnn_partials(m1)
        dt = (t[s] - t[s - 1]).reshape(1, 1)
        temp, xii, xi, m1 = _a_call(dt, p2[0], p2[1], temp, h,
                                    wih_t, whh_t, bih, bhh, w1t, b1r)
        h = xi
        outs.append(temp)
        douts.append(xii)

    p2 = gnn_partials(m1)
    (xii,) = _c_call(p2[0], p2[1])
    douts.append(xii)

    output = jnp.stack(outs, axis=0)
    doutput = jnp.stack(douts, axis=0)
    return jnp.concatenate([output, doutput], axis=-1)
